# Initial kernel scaffold; baseline (speedup 1.0000x reference)
import jax
import jax.numpy as jnp
from jax.experimental import pallas as pl


def kernel(q_xyz, q_feat, kv_xyz, kv_feat, Wp, bp, Wq, bq, Wk, bk, Wv, bv, Wo, bo, g1, be1, g2, be2, W1, b1, W2, b2):
    raise NotImplementedError("write your pallas kernel here")


# speedup vs baseline: 16.6140x; 16.6140x over previous
import functools
import math

import jax
import jax.numpy as jnp
from jax import lax
from jax.experimental import pallas as pl
from jax.experimental.pallas import tpu as pltpu
from jax.experimental.pallas import tpu_sc as plsc

B, N, M, C, H, K = 4, 2048, 4096, 256, 8, 8
RADIUS = 0.2
DH = C // H
FF = 4 * C

TQ = 256
TA = 128
GCHUNK = 128


def _proj_kv_body(xyz_ref, feat_ref, Wp_ref, bp_ref, Wk_ref, bk_ref,
                  Wv_ref, bv_ref, kk_ref, vv_ref):
    feat = feat_ref[...]
    pos = (xyz_ref[:, 0:1] * Wp_ref[0:1, :]
           + xyz_ref[:, 1:2] * Wp_ref[1:2, :]
           + xyz_ref[:, 2:3] * Wp_ref[2:3, :]) + bp_ref[...]
    tk = feat + pos
    kk_ref[...] = lax.dot_general(tk, Wk_ref[...], (((1,), (0,)), ((), ())),
                                  preferred_element_type=jnp.float32) + bk_ref[...]
    vv_ref[...] = lax.dot_general(feat, Wv_ref[...], (((1,), (0,)), ((), ())),
                                  preferred_element_type=jnp.float32) + bv_ref[...]


def _proj_q_body(xyz_ref, feat_ref, Wp_ref, bp_ref, Wq_ref, bq_ref,
                 Wk_ref, bk_ref, Wv_ref, bv_ref, q0_ref, kq_ref, vq_ref):
    feat = feat_ref[...]
    pos = (xyz_ref[:, 0:1] * Wp_ref[0:1, :]
           + xyz_ref[:, 1:2] * Wp_ref[1:2, :]
           + xyz_ref[:, 2:3] * Wp_ref[2:3, :]) + bp_ref[...]
    qk_in = feat + pos
    q0_ref[...] = lax.dot_general(qk_in, Wq_ref[...], (((1,), (0,)), ((), ())),
                                  preferred_element_type=jnp.float32) + bq_ref[...]
    kq_ref[...] = lax.dot_general(qk_in, Wk_ref[...], (((1,), (0,)), ((), ())),
                                  preferred_element_type=jnp.float32) + bk_ref[...]
    vq_ref[...] = lax.dot_general(feat, Wv_ref[...], (((1,), (0,)), ((), ())),
                                  preferred_element_type=jnp.float32) + bv_ref[...]


def _topk_body(q_ref, kt_ref, idx_ref, inv_ref, d2_ref):
    b = pl.program_id(0)
    inf = jnp.float32(jnp.inf)

    qx = q_ref[0, :, 0:1]
    qy = q_ref[0, :, 1:2]
    qz = q_ref[0, :, 2:3]
    kx = kt_ref[0, 0:1, :]
    ky = kt_ref[0, 1:2, :]
    kz = kt_ref[0, 2:3, :]
    qb = q_ref[0].astype(jnp.bfloat16)
    kb = kt_ref[0].astype(jnp.bfloat16)
    cross = lax.dot_general(qb, kb, (((1,), (0,)), ((), ())),
                            preferred_element_type=jnp.float32)
    qn = qx * qx + qy * qy + qz * qz
    kn = kx * kx + ky * ky + kz * kz
    d2 = (qn + kn) - 2.0 * cross
    dist = jnp.sqrt(jnp.maximum(d2, 1e-12))
    d2_ref[...] = jnp.where(dist <= RADIUS, dist, inf)

    iota = lax.broadcasted_iota(jnp.int32, (1, M), 1)
    idx_cols = []
    inv_cols = []
    for _ in range(K):
        d2s = d2_ref[...]
        m = jnp.min(d2s, axis=1, keepdims=True)
        cand = jnp.where(d2s <= m, iota, jnp.int32(M))
        sel = jnp.min(cand, axis=1, keepdims=True)
        invalid = m == inf
        sel = jnp.where(invalid, 0, sel)
        idx_cols.append(sel + b * M)
        inv_cols.append(invalid.astype(jnp.float32))
        d2_ref[...] = jnp.where(iota == sel, inf, d2s)
    idx_ref[0] = jnp.concatenate(idx_cols, axis=1)
    inv_ref[0] = jnp.concatenate(inv_cols, axis=1)


def _attn_body(q0_ref, kq_ref, vq_ref, kg_ref, vg_ref, inv_ref, qf_ref,
               Wo_ref, bo_ref, g1_ref, be1_ref, g2_ref, be2_ref,
               W1_ref, b1_ref, W2_ref, b2_ref, out_ref):
    scale = jnp.float32(1.0 / math.sqrt(DH))
    rows = lax.broadcasted_iota(jnp.int32, (C, C), 0)
    cols = lax.broadcasted_iota(jnp.int32, (C, C), 1)
    seg = (rows // DH == cols // DH).astype(jnp.float32)

    def head_dot(a, b):
        return lax.dot_general(a * b, seg, (((1,), (0,)), ((), ())),
                               preferred_element_type=jnp.float32) * scale

    q0 = q0_ref[...]
    s = [head_dot(q0, kq_ref[...])]
    for j in range(K):
        sj = head_dot(q0, kg_ref[:, j, :])
        sj = jnp.where(inv_ref[:, j:j + 1] > 0.5, jnp.float32(-1e9), sj)
        s.append(sj)
    mx = s[0]
    for j in range(1, K + 1):
        mx = jnp.maximum(mx, s[j])
    e0 = jnp.exp(s[0] - mx)
    denom = e0
    acc = e0 * vq_ref[...]
    for j in range(K):
        ej = jnp.exp(s[j + 1] - mx)
        denom = denom + ej
        acc = acc + ej * vg_ref[:, j, :]
    out0 = acc / denom

    proj = lax.dot_general(out0, Wo_ref[...], (((1,), (0,)), ((), ())),
                           preferred_element_type=jnp.float32) + bo_ref[...]
    x = qf_ref[...] + proj
    mu = jnp.mean(x, axis=1, keepdims=True)
    cxt = x - mu
    var = jnp.mean(cxt * cxt, axis=1, keepdims=True)
    x = cxt / jnp.sqrt(var + 1e-5) * g1_ref[...] + be1_ref[...]

    h1 = lax.dot_general(x, W1_ref[...], (((1,), (0,)), ((), ())),
                         preferred_element_type=jnp.float32) + b1_ref[...]
    h1 = jnp.maximum(h1, 0.0)
    ff = lax.dot_general(h1, W2_ref[...], (((1,), (0,)), ((), ())),
                         preferred_element_type=jnp.float32) + b2_ref[...]
    y = x + ff
    mu2 = jnp.mean(y, axis=1, keepdims=True)
    cy = y - mu2
    var2 = jnp.mean(cy * cy, axis=1, keepdims=True)
    y = cy / jnp.sqrt(var2 + 1e-5) * g2_ref[...] + be2_ref[...]
    out_ref[...] = y + qf_ref[...]


def _make_sc_gather():
    info = plsc.get_sparse_core_info()
    nw = info.num_cores * info.num_subcores
    total = B * N * K
    per_w = total // nw
    chunks = per_w // GCHUNK
    mesh = plsc.VectorSubcoreMesh(core_axis_name="c", subcore_axis_name="s")

    @functools.partial(
        pl.kernel, mesh=mesh,
        out_type=[jax.ShapeDtypeStruct((total, C), jnp.float32),
                  jax.ShapeDtypeStruct((total, C), jnp.float32)],
        scratch_types=[pltpu.VMEM((GCHUNK,), jnp.int32),
                       pltpu.VMEM((GCHUNK, C), jnp.float32),
                       pltpu.VMEM((GCHUNK, C), jnp.float32),
                       pltpu.SemaphoreType.DMA],
    )
    def gather(ktab_hbm, vtab_hbm, idx_hbm, kout_hbm, vout_hbm,
               idx_v, krows_v, vrows_v, sem):
        wid = lax.axis_index("s") * info.num_cores + lax.axis_index("c")

        def body(c, _):
            base = wid * per_w + c * GCHUNK
            pltpu.sync_copy(idx_hbm.at[pl.ds(base, GCHUNK)], idx_v)
            pltpu.async_copy(ktab_hbm.at[idx_v], krows_v, sem).wait()
            pltpu.async_copy(vtab_hbm.at[idx_v], vrows_v, sem).wait()
            pltpu.sync_copy(krows_v, kout_hbm.at[pl.ds(base, GCHUNK)])
            pltpu.sync_copy(vrows_v, vout_hbm.at[pl.ds(base, GCHUNK)])
            return ()

        lax.fori_loop(0, chunks, body, ())

    return gather


_sc_gather = None


def _full(shape):
    return pl.BlockSpec(shape, lambda *a: (0,) * len(shape))


def kernel(q_xyz, q_feat, kv_xyz, kv_feat, Wp, bp, Wq, bq, Wk, bk, Wv, bv,
           Wo, bo, g1, be1, g2, be2, W1, b1, W2, b2):
    global _sc_gather
    if _sc_gather is None:
        _sc_gather = _make_sc_gather()

    bp2, bq2, bk2, bv2, bo2 = (x.reshape(1, C) for x in (bp, bq, bk, bv, bo))
    g1r, be1r, g2r, be2r, b2r = (x.reshape(1, C) for x in (g1, be1, g2, be2, b2))
    b1r = b1.reshape(1, FF)

    kv_flat_xyz = kv_xyz.reshape(B * M, 3)
    kv_flat_feat = kv_feat.reshape(B * M, C)
    q_flat_xyz = q_xyz.reshape(B * N, 3)
    q_flat_feat = q_feat.reshape(B * N, C)

    kkv, vkv = pl.pallas_call(
        _proj_kv_body,
        grid=(B * M // TQ,),
        in_specs=[pl.BlockSpec((TQ, 3), lambda i: (i, 0)),
                  pl.BlockSpec((TQ, C), lambda i: (i, 0)),
                  _full((3, C)), _full((1, C)),
                  _full((C, C)), _full((1, C)),
                  _full((C, C)), _full((1, C))],
        out_specs=[pl.BlockSpec((TQ, C), lambda i: (i, 0)),
                   pl.BlockSpec((TQ, C), lambda i: (i, 0))],
        out_shape=[jax.ShapeDtypeStruct((B * M, C), jnp.float32),
                   jax.ShapeDtypeStruct((B * M, C), jnp.float32)],
    )(kv_flat_xyz, kv_flat_feat, Wp, bp2, Wk, bk2, Wv, bv2)

    q0, kq, vq = pl.pallas_call(
        _proj_q_body,
        grid=(B * N // TQ,),
        in_specs=[pl.BlockSpec((TQ, 3), lambda i: (i, 0)),
                  pl.BlockSpec((TQ, C), lambda i: (i, 0)),
                  _full((3, C)), _full((1, C)),
                  _full((C, C)), _full((1, C)),
                  _full((C, C)), _full((1, C)),
                  _full((C, C)), _full((1, C))],
        out_specs=[pl.BlockSpec((TQ, C), lambda i: (i, 0)),
                   pl.BlockSpec((TQ, C), lambda i: (i, 0)),
                   pl.BlockSpec((TQ, C), lambda i: (i, 0))],
        out_shape=[jax.ShapeDtypeStruct((B * N, C), jnp.float32),
                   jax.ShapeDtypeStruct((B * N, C), jnp.float32),
                   jax.ShapeDtypeStruct((B * N, C), jnp.float32)],
    )(q_flat_xyz, q_flat_feat, Wp, bp2, Wq, bq2, Wk, bk2, Wv, bv2)

    kv_xyz_t = kv_xyz.transpose(0, 2, 1)
    idx, invalid = pl.pallas_call(
        _topk_body,
        grid=(B, N // TQ),
        in_specs=[pl.BlockSpec((1, TQ, 3), lambda b, i: (b, i, 0)),
                  pl.BlockSpec((1, 3, M), lambda b, i: (b, 0, 0))],
        out_specs=[pl.BlockSpec((1, TQ, K), lambda b, i: (b, i, 0)),
                   pl.BlockSpec((1, TQ, K), lambda b, i: (b, i, 0))],
        out_shape=[jax.ShapeDtypeStruct((B, N, K), jnp.int32),
                   jax.ShapeDtypeStruct((B, N, K), jnp.float32)],
        scratch_shapes=[pltpu.VMEM((TQ, M), jnp.float32)],
    )(q_xyz, kv_xyz_t)

    flat_idx = idx.reshape(B * N * K)
    kg, vg = _sc_gather(kkv, vkv, flat_idx)
    kg = kg.reshape(B * N, K, C)
    vg = vg.reshape(B * N, K, C)
    inv2 = invalid.reshape(B * N, K)

    out = pl.pallas_call(
        _attn_body,
        grid=(B * N // TA,),
        in_specs=[pl.BlockSpec((TA, C), lambda i: (i, 0)),
                  pl.BlockSpec((TA, C), lambda i: (i, 0)),
                  pl.BlockSpec((TA, C), lambda i: (i, 0)),
                  pl.BlockSpec((TA, K, C), lambda i: (i, 0, 0)),
                  pl.BlockSpec((TA, K, C), lambda i: (i, 0, 0)),
                  pl.BlockSpec((TA, K), lambda i: (i, 0)),
                  pl.BlockSpec((TA, C), lambda i: (i, 0)),
                  _full((C, C)), _full((1, C)),
                  _full((1, C)), _full((1, C)),
                  _full((1, C)), _full((1, C)),
                  _full((C, FF)), _full((1, FF)),
                  _full((FF, C)), _full((1, C))],
        out_specs=pl.BlockSpec((TA, C), lambda i: (i, 0)),
        out_shape=jax.ShapeDtypeStruct((B * N, C), jnp.float32),
    )(q0, kq, vq, kg, vg, inv2, q_flat_feat,
      Wo, bo2, g1r, be1r, g2r, be2r, W1, b1r, W2, b2r)

    return out.reshape(B, N, C)


# speedup vs baseline: 22.6800x; 1.3651x over previous
import functools
import math

import jax
import jax.numpy as jnp
from jax import lax
from jax.experimental import pallas as pl
from jax.experimental.pallas import tpu as pltpu
from jax.experimental.pallas import tpu_sc as plsc

B, N, M, C, H, K = 4, 2048, 4096, 256, 8, 8
RADIUS = 0.2
DH = C // H
FF = 4 * C

TQ = 256
TT = 512
TA = 256
GCHUNK = 128


HC = C // 2

def _pack_pair(x):
    xb = x.astype(jnp.bfloat16).astype(jnp.float32)
    be = lax.bitcast_convert_type(xb[:, :HC], jnp.int32)
    bo = lax.bitcast_convert_type(xb[:, HC:], jnp.int32)
    return lax.shift_right_logical(be, 16) | (bo & jnp.int32(-65536))


def _unpack_pair(w):
    e = lax.bitcast_convert_type(lax.shift_left(w, 16), jnp.float32)
    o = lax.bitcast_convert_type(w & jnp.int32(-65536), jnp.float32)
    return e, o


def _proj_kv_body(xyz_ref, feat_ref, Wp_ref, bp_ref, Wk_ref, bk_ref,
                  Wv_ref, bv_ref, kv_ref):
    feat = feat_ref[...]
    pos = (xyz_ref[:, 0:1] * Wp_ref[0:1, :]
           + xyz_ref[:, 1:2] * Wp_ref[1:2, :]
           + xyz_ref[:, 2:3] * Wp_ref[2:3, :]) + bp_ref[...]
    tk = (feat + pos).astype(jnp.bfloat16)
    fb = feat.astype(jnp.bfloat16)
    kk = lax.dot_general(tk, Wk_ref[...].astype(jnp.bfloat16),
                         (((1,), (0,)), ((), ())),
                         preferred_element_type=jnp.float32) + bk_ref[...]
    vv = lax.dot_general(fb, Wv_ref[...].astype(jnp.bfloat16),
                         (((1,), (0,)), ((), ())),
                         preferred_element_type=jnp.float32) + bv_ref[...]
    kv_ref[:, :HC] = _pack_pair(kk)
    kv_ref[:, HC:] = _pack_pair(vv)


def _proj_q_body(xyz_ref, feat_ref, Wp_ref, bp_ref, Wq_ref, bq_ref,
                 Wk_ref, bk_ref, Wv_ref, bv_ref, q0_ref, kq_ref, vq_ref):
    feat = feat_ref[...]
    pos = (xyz_ref[:, 0:1] * Wp_ref[0:1, :]
           + xyz_ref[:, 1:2] * Wp_ref[1:2, :]
           + xyz_ref[:, 2:3] * Wp_ref[2:3, :]) + bp_ref[...]
    qk_in = (feat + pos).astype(jnp.bfloat16)
    fb = feat.astype(jnp.bfloat16)
    q0_ref[...] = lax.dot_general(qk_in, Wq_ref[...].astype(jnp.bfloat16),
                                  (((1,), (0,)), ((), ())),
                                  preferred_element_type=jnp.float32) + bq_ref[...]
    kq_ref[...] = lax.dot_general(qk_in, Wk_ref[...].astype(jnp.bfloat16),
                                  (((1,), (0,)), ((), ())),
                                  preferred_element_type=jnp.float32) + bk_ref[...]
    vq_ref[...] = lax.dot_general(fb, Wv_ref[...].astype(jnp.bfloat16),
                                  (((1,), (0,)), ((), ())),
                                  preferred_element_type=jnp.float32) + bv_ref[...]


def _topk_body(q_ref, kt_ref, idx_ref, inv_ref, d2_ref):
    b = pl.program_id(0)
    inf = jnp.float32(jnp.inf)

    qx = q_ref[0, :, 0:1]
    qy = q_ref[0, :, 1:2]
    qz = q_ref[0, :, 2:3]
    kx = kt_ref[0, 0:1, :]
    ky = kt_ref[0, 1:2, :]
    kz = kt_ref[0, 2:3, :]
    qb = q_ref[0].astype(jnp.bfloat16)
    kb = kt_ref[0].astype(jnp.bfloat16)
    cross = lax.dot_general(qb, kb, (((1,), (0,)), ((), ())),
                            preferred_element_type=jnp.float32)
    qn = qx * qx + qy * qy + qz * qz
    kn = kx * kx + ky * ky + kz * kz
    d2 = (qn + kn) - 2.0 * cross
    dist = jnp.sqrt(jnp.maximum(d2, 1e-12))
    d2m = jnp.where(dist <= RADIUS, dist, inf)
    d2_ref[...] = d2m

    iota_f = lax.broadcasted_iota(jnp.int32, (1, M), 1).astype(jnp.float32)
    idx_cols = []
    inv_cols = []
    m = jnp.min(d2m, axis=1, keepdims=True)
    for k in range(K):
        d2s = d2_ref[...]
        cand = jnp.where(d2s <= m, iota_f, jnp.float32(M))
        self_f = jnp.min(cand, axis=1, keepdims=True)
        invalid = m == inf
        sel = jnp.where(invalid, 0, self_f.astype(jnp.int32))
        idx_cols.append(sel + b * M)
        inv_cols.append(invalid.astype(jnp.float32))
        if k < K - 1:
            d2_new = jnp.where(iota_f == self_f, inf, d2s)
            d2_ref[...] = d2_new
            m = jnp.min(d2_new, axis=1, keepdims=True)
    idx_ref[0] = jnp.concatenate(idx_cols, axis=1)
    inv_ref[0] = jnp.concatenate(inv_cols, axis=1)


def _attn_body(q0_ref, kq_ref, vq_ref, kvg_ref, inv_ref, qf_ref,
               Wo_ref, bo_ref, g1_ref, be1_ref, g2_ref, be2_ref,
               W1_ref, b1_ref, W2_ref, b2_ref, out_ref):
    scale = jnp.float32(1.0 / math.sqrt(DH))
    rows = lax.broadcasted_iota(jnp.int32, (HC, HC), 0)
    cols = lax.broadcasted_iota(jnp.int32, (HC, HC), 1)
    seg = (rows // (DH // 2) == cols // (DH // 2)).astype(jnp.bfloat16)

    def head_dot(prod):
        return lax.dot_general(prod.astype(jnp.bfloat16), seg,
                               (((1,), (0,)), ((), ())),
                               preferred_element_type=jnp.float32) * scale

    q0e = q0_ref[:, :HC]
    q0o = q0_ref[:, HC:]
    s = [head_dot(q0e * kq_ref[:, :HC] + q0o * kq_ref[:, HC:])]
    vs = []
    for j in range(K):
        ke, ko = _unpack_pair(kvg_ref[j, :, :HC])
        vs.append(_unpack_pair(kvg_ref[j, :, HC:]))
        sj = head_dot(q0e * ke + q0o * ko)
        sj = jnp.where(inv_ref[:, j:j + 1] > 0.5, jnp.float32(-1e9), sj)
        s.append(sj)
    mx = s[0]
    for j in range(1, K + 1):
        mx = jnp.maximum(mx, s[j])
    e0 = jnp.exp(s[0] - mx)
    denom = e0
    acc_e = e0 * vq_ref[:, :HC]
    acc_o = e0 * vq_ref[:, HC:]
    for j in range(K):
        ej = jnp.exp(s[j + 1] - mx)
        denom = denom + ej
        acc_e = acc_e + ej * vs[j][0]
        acc_o = acc_o + ej * vs[j][1]
    out0 = jnp.concatenate([acc_e / denom, acc_o / denom], axis=1)

    proj = lax.dot_general(out0.astype(jnp.bfloat16),
                           Wo_ref[...].astype(jnp.bfloat16),
                           (((1,), (0,)), ((), ())),
                           preferred_element_type=jnp.float32) + bo_ref[...]
    x = qf_ref[...] + proj
    mu = jnp.mean(x, axis=1, keepdims=True)
    cxt = x - mu
    var = jnp.mean(cxt * cxt, axis=1, keepdims=True)
    x = cxt / jnp.sqrt(var + 1e-5) * g1_ref[...] + be1_ref[...]

    h1 = lax.dot_general(x.astype(jnp.bfloat16), W1_ref[...].astype(jnp.bfloat16),
                         (((1,), (0,)), ((), ())),
                         preferred_element_type=jnp.float32) + b1_ref[...]
    h1 = jnp.maximum(h1, 0.0)
    ff = lax.dot_general(h1.astype(jnp.bfloat16), W2_ref[...].astype(jnp.bfloat16),
                         (((1,), (0,)), ((), ())),
                         preferred_element_type=jnp.float32) + b2_ref[...]
    y = x + ff
    mu2 = jnp.mean(y, axis=1, keepdims=True)
    cy = y - mu2
    var2 = jnp.mean(cy * cy, axis=1, keepdims=True)
    y = cy / jnp.sqrt(var2 + 1e-5) * g2_ref[...] + be2_ref[...]
    out_ref[...] = y + qf_ref[...]


def _make_sc_gather():
    info = plsc.get_sparse_core_info()
    nw = info.num_cores * info.num_subcores
    total = B * N * K
    per_w = total // nw
    chunks = per_w // GCHUNK
    mesh = plsc.VectorSubcoreMesh(core_axis_name="c", subcore_axis_name="s")

    @functools.partial(
        pl.kernel, mesh=mesh,
        out_type=jax.ShapeDtypeStruct((total, C), jnp.int32),
        scratch_types=[pltpu.VMEM((GCHUNK,), jnp.int32),
                       pltpu.VMEM((GCHUNK,), jnp.int32),
                       pltpu.VMEM((GCHUNK, C), jnp.int32),
                       pltpu.VMEM((GCHUNK, C), jnp.int32),
                       pltpu.SemaphoreType.DMA,
                       pltpu.SemaphoreType.DMA],
    )
    def gather(tab_hbm, idx_hbm, out_hbm,
               idx_v0, idx_v1, rows_v0, rows_v1, sem0, sem1):
        wid = lax.axis_index("s") * info.num_cores + lax.axis_index("c")
        idx_v = (idx_v0, idx_v1)
        rows_v = (rows_v0, rows_v1)
        sem = (sem0, sem1)

        def start(c, buf):
            base = wid * per_w + c * GCHUNK
            pltpu.sync_copy(idx_hbm.at[pl.ds(base, GCHUNK)], idx_v[buf])
            return pltpu.async_copy(tab_hbm.at[idx_v[buf]], rows_v[buf], sem[buf])

        cp = start(0, 0)
        for c in range(chunks):
            cur = c % 2
            nxt = None
            if c + 1 < chunks:
                nxt = start(c + 1, 1 - cur)
            cp.wait()
            base = wid * per_w + c * GCHUNK
            pltpu.sync_copy(rows_v[cur], out_hbm.at[pl.ds(base, GCHUNK)])
            cp = nxt

    return gather


_sc_gather = None


def _full(shape):
    return pl.BlockSpec(shape, lambda *a: (0,) * len(shape))


def kernel(q_xyz, q_feat, kv_xyz, kv_feat, Wp, bp, Wq, bq, Wk, bk, Wv, bv,
           Wo, bo, g1, be1, g2, be2, W1, b1, W2, b2):
    global _sc_gather
    if _sc_gather is None:
        _sc_gather = _make_sc_gather()

    perm = jnp.concatenate([jnp.arange(0, C, 2), jnp.arange(1, C, 2)])
    Wq = Wq[:, perm]
    Wk = Wk[:, perm]
    Wv = Wv[:, perm]
    Wo = Wo[perm, :]
    bq = bq[perm]
    bk = bk[perm]
    bv = bv[perm]
    bp2, bq2, bk2, bv2, bo2 = (x.reshape(1, C) for x in (bp, bq, bk, bv, bo))
    g1r, be1r, g2r, be2r, b2r = (x.reshape(1, C) for x in (g1, be1, g2, be2, b2))
    b1r = b1.reshape(1, FF)

    kv_flat_xyz = kv_xyz.reshape(B * M, 3)
    kv_flat_feat = kv_feat.reshape(B * M, C)
    q_flat_xyz = q_xyz.reshape(B * N, 3)
    q_flat_feat = q_feat.reshape(B * N, C)

    kvtab = pl.pallas_call(
        _proj_kv_body,
        grid=(B * M // TQ,),
        in_specs=[pl.BlockSpec((TQ, 3), lambda i: (i, 0)),
                  pl.BlockSpec((TQ, C), lambda i: (i, 0)),
                  _full((3, C)), _full((1, C)),
                  _full((C, C)), _full((1, C)),
                  _full((C, C)), _full((1, C))],
        out_specs=pl.BlockSpec((TQ, C), lambda i: (i, 0)),
        out_shape=jax.ShapeDtypeStruct((B * M, C), jnp.int32),
    )(kv_flat_xyz, kv_flat_feat, Wp, bp2, Wk, bk2, Wv, bv2)

    q0, kq, vq = pl.pallas_call(
        _proj_q_body,
        grid=(B * N // TQ,),
        in_specs=[pl.BlockSpec((TQ, 3), lambda i: (i, 0)),
                  pl.BlockSpec((TQ, C), lambda i: (i, 0)),
                  _full((3, C)), _full((1, C)),
                  _full((C, C)), _full((1, C)),
                  _full((C, C)), _full((1, C)),
                  _full((C, C)), _full((1, C))],
        out_specs=[pl.BlockSpec((TQ, C), lambda i: (i, 0)),
                   pl.BlockSpec((TQ, C), lambda i: (i, 0)),
                   pl.BlockSpec((TQ, C), lambda i: (i, 0))],
        out_shape=[jax.ShapeDtypeStruct((B * N, C), jnp.float32),
                   jax.ShapeDtypeStruct((B * N, C), jnp.float32),
                   jax.ShapeDtypeStruct((B * N, C), jnp.float32)],
    )(q_flat_xyz, q_flat_feat, Wp, bp2, Wq, bq2, Wk, bk2, Wv, bv2)

    kv_xyz_t = kv_xyz.transpose(0, 2, 1)
    idx, invalid = pl.pallas_call(
        _topk_body,
        grid=(B, N // TT),
        in_specs=[pl.BlockSpec((1, TT, 3), lambda b, i: (b, i, 0)),
                  pl.BlockSpec((1, 3, M), lambda b, i: (b, 0, 0))],
        out_specs=[pl.BlockSpec((1, TT, K), lambda b, i: (b, i, 0)),
                   pl.BlockSpec((1, TT, K), lambda b, i: (b, i, 0))],
        out_shape=[jax.ShapeDtypeStruct((B, N, K), jnp.int32),
                   jax.ShapeDtypeStruct((B, N, K), jnp.float32)],
        scratch_shapes=[pltpu.VMEM((TT, M), jnp.float32)],
    )(q_xyz, kv_xyz_t)

    flat_idx = idx.reshape(B * N, K).T.reshape(B * N * K)
    kvg = _sc_gather(kvtab, flat_idx)
    kvg = kvg.reshape(K, B * N, C)
    inv2 = invalid.reshape(B * N, K)

    out = pl.pallas_call(
        _attn_body,
        grid=(B * N // TA,),
        in_specs=[pl.BlockSpec((TA, C), lambda i: (i, 0)),
                  pl.BlockSpec((TA, C), lambda i: (i, 0)),
                  pl.BlockSpec((TA, C), lambda i: (i, 0)),
                  pl.BlockSpec((K, TA, C), lambda i: (0, i, 0)),
                  pl.BlockSpec((TA, K), lambda i: (i, 0)),
                  pl.BlockSpec((TA, C), lambda i: (i, 0)),
                  _full((C, C)), _full((1, C)),
                  _full((1, C)), _full((1, C)),
                  _full((1, C)), _full((1, C)),
                  _full((C, FF)), _full((1, FF)),
                  _full((FF, C)), _full((1, C))],
        out_specs=pl.BlockSpec((TA, C), lambda i: (i, 0)),
        out_shape=jax.ShapeDtypeStruct((B * N, C), jnp.float32),
    )(q0, kq, vq, kvg, inv2, q_flat_feat,
      Wo, bo2, g1r, be1r, g2r, be2r, W1, b1r, W2, b2r)

    return out.reshape(B, N, C)
